# baseline (speedup 1.0000x reference)
import functools

import jax
import jax.numpy as jnp
from jax import lax
from jax.experimental import pallas as pl
from jax.experimental.pallas import tpu as pltpu
from jax.experimental.pallas import tpu_sc as plsc

N = 10000
E = 160000
NC = 2
NS = 16
EPAD = 163840
EPT = EPAD // NS
CHE = 512
NPAD = 10240
HALF = 128
PH = 5120
ZROW = 2 * N
TROWS = 20480
LANES = 128
BM = 1000


def _mesh():
    return plsc.VectorSubcoreMesh(
        core_axis_name="c", subcore_axis_name="s", num_cores=NC, num_subcores=NS
    )


def _make_degree_kernel():
    edges = EPAD // (NC * NS)

    @functools.partial(
        pl.kernel,
        out_type=jax.ShapeDtypeStruct((NC * NPAD,), jnp.float32),
        mesh=_mesh(),
        scratch_types=[
            pltpu.VMEM((edges,), jnp.int32),
            pltpu.VMEM((edges,), jnp.float32),
            pltpu.VMEM((NPAD // NS,), jnp.float32),
            pltpu.VMEM_SHARED((NPAD,), jnp.float32),
        ],
    )
    def deg_kernel(dstp_hbm, zo_hbm, out_hbm, idxd_v, ones_v, stage_v, acc_sh):
        cid = lax.axis_index("c")
        sid = lax.axis_index("s")
        t = cid * NS + sid
        seg = NPAD // NS

        pltpu.sync_copy(zo_hbm.at[pl.ds(0, seg)], stage_v)
        pltpu.sync_copy(zo_hbm.at[pl.ds(seg, edges)], ones_v)
        pltpu.sync_copy(stage_v, acc_sh.at[pl.ds(sid * seg, seg)])
        pltpu.sync_copy(dstp_hbm.at[pl.ds(edges * t, edges)], idxd_v)
        plsc.subcore_barrier()
        pltpu.sync_copy(ones_v, acc_sh.at[idxd_v], add=True)
        plsc.subcore_barrier()
        pltpu.sync_copy(acc_sh.at[pl.ds(sid * seg, seg)], stage_v)
        pltpu.sync_copy(stage_v, out_hbm.at[pl.ds(cid * NPAD + sid * seg, seg)])

    return deg_kernel


def _make_prop_kernel():

    @functools.partial(
        pl.kernel,
        out_type=jax.ShapeDtypeStruct((NC * 2 * PH, HALF), jnp.float32),
        mesh=_mesh(),
        scratch_types=[
            pltpu.VMEM((EPT,), jnp.int32),
            pltpu.VMEM((EPT,), jnp.int32),
            pltpu.VMEM((CHE, HALF), jnp.float32),
            pltpu.VMEM_SHARED((PH, HALF), jnp.float32),
            pltpu.SemaphoreType.DMA,
        ],
    )
    def prop_kernel(
        xc2_hbm, srcp4_hbm, dstp2_hbm, out_hbm, idxs_v, idxd_v, rows_v, acc_sh, sem
    ):
        cid = lax.axis_index("c")
        sid = lax.axis_index("s")

        def zfill(i, carry):
            for k in range(HALF // 16):
                rows_v[i, pl.ds(k * 16, 16)] = jnp.zeros((16,), jnp.float32)
            return carry

        lax.fori_loop(0, LANES, zfill, 0)

        for p in range(2):
            @pl.when(sid < 8)
            def _zero():
                for j in range(PH // 8 // LANES):
                    pltpu.sync_copy(
                        rows_v.at[pl.ds(0, LANES)],
                        acc_sh.at[pl.ds(sid * (PH // 8) + j * LANES, LANES)],
                    )
            pltpu.sync_copy(
                srcp4_hbm.at[pl.ds((cid * 2 + p) * EPAD + sid * EPT, EPT)], idxs_v
            )
            pltpu.sync_copy(dstp2_hbm.at[pl.ds(p * EPAD + sid * EPT, EPT)], idxd_v)
            plsc.subcore_barrier()

            def body(j, carry):
                pltpu.async_copy(
                    xc2_hbm.at[idxs_v.at[pl.ds(j * CHE, CHE)]], rows_v, sem
                ).wait()
                pltpu.sync_copy(
                    rows_v, acc_sh.at[idxd_v.at[pl.ds(j * CHE, CHE)]], add=True
                )
                return carry

            lax.fori_loop(0, EPT // CHE, body, 0)
            plsc.subcore_barrier()

            @pl.when(sid < 8)
            def _writeback():
                for j in range(PH // 8 // LANES):
                    pltpu.sync_copy(
                        acc_sh.at[pl.ds(sid * (PH // 8) + j * LANES, LANES)],
                        rows_v.at[pl.ds(0, LANES)],
                    )
                    pltpu.sync_copy(
                        rows_v.at[pl.ds(0, LANES)],
                        out_hbm.at[
                            pl.ds(
                                cid * 2 * PH
                                + p * PH
                                + sid * (PH // 8)
                                + j * LANES,
                                LANES,
                            )
                        ],
                    )

            if p == 0:
                lax.fori_loop(0, LANES, zfill, 0)
                plsc.subcore_barrier()

    return prop_kernel


def _dv_block(degt_blk):
    return lax.rsqrt(degt_blk[:, 0:1] + degt_blk[:, 1:2] + 1.0)


def _tc_prep(degt, x):
    def body(degt_ref, x_ref, out_ref):
        dv = _dv_block(degt_ref[...])
        xb = x_ref[...]
        out_ref[0] = xb[:, :HALF] * dv
        out_ref[1] = xb[:, HALF:] * dv

    return pl.pallas_call(
        body,
        grid=(N // BM,),
        in_specs=[
            pl.BlockSpec((BM, 2), lambda i: (i, 0)),
            pl.BlockSpec((BM, 2 * HALF), lambda i: (i, 0)),
        ],
        out_specs=pl.BlockSpec((2, BM, HALF), lambda i: (0, i, 0)),
        out_shape=jax.ShapeDtypeStruct((2, N, HALF), jnp.float32),
    )(degt, x)


def _tc_main(degt, S1, XC, W1, b1, W2):
    def body(degt_ref, s1_ref, xc_ref, w1_ref, b1_ref, w2_ref, out_ref):
        dv = _dv_block(degt_ref[...])
        p = jnp.concatenate(
            [(s1_ref[0] + xc_ref[0]) * dv, (s1_ref[1] + xc_ref[1]) * dv], axis=1
        )
        h = jnp.dot(p, w1_ref[...], preferred_element_type=jnp.float32)
        h = jnp.maximum(h + b1_ref[...], 0.0)
        y = jnp.dot(h, w2_ref[...], preferred_element_type=jnp.float32)
        out_ref[0] = y[:, :HALF] * dv
        out_ref[1] = y[:, HALF:] * dv

    return pl.pallas_call(
        body,
        grid=(N // BM,),
        in_specs=[
            pl.BlockSpec((BM, 2), lambda i: (i, 0)),
            pl.BlockSpec((2, BM, HALF), lambda i: (0, i, 0)),
            pl.BlockSpec((2, BM, HALF), lambda i: (0, i, 0)),
            pl.BlockSpec(W1.shape, lambda i: (0, 0)),
            pl.BlockSpec((1, W1.shape[1]), lambda i: (0, 0)),
            pl.BlockSpec(W2.shape, lambda i: (0, 0)),
        ],
        out_specs=pl.BlockSpec((2, BM, HALF), lambda i: (0, i, 0)),
        out_shape=jax.ShapeDtypeStruct((2, N, HALF), jnp.float32),
    )(degt, S1, XC, W1, b1, W2)


def _tc_final(degt, S2, Y2, b2):
    def body(degt_ref, s2_ref, y2_ref, b2_ref, out_ref):
        dv = _dv_block(degt_ref[...])
        out_ref[...] = (
            jnp.concatenate(
                [(s2_ref[0] + y2_ref[0]) * dv, (s2_ref[1] + y2_ref[1]) * dv], axis=1
            )
            + b2_ref[...]
        )

    return pl.pallas_call(
        body,
        grid=(N // BM,),
        in_specs=[
            pl.BlockSpec((BM, 2), lambda i: (i, 0)),
            pl.BlockSpec((2, BM, HALF), lambda i: (0, i, 0)),
            pl.BlockSpec((2, BM, HALF), lambda i: (0, i, 0)),
            pl.BlockSpec((1, 2 * HALF), lambda i: (0, 0)),
        ],
        out_specs=pl.BlockSpec((BM, 2 * HALF), lambda i: (i, 0)),
        out_shape=jax.ShapeDtypeStruct((N, 2 * HALF), jnp.float32),
    )(degt, S2, Y2, b2)


def kernel(x, edge_index, W1, b1, W2, b2):
    src = edge_index[0].astype(jnp.int32)
    dst = edge_index[1].astype(jnp.int32)
    epad = EPAD - E
    dstd = jnp.concatenate([dst, jnp.full((epad,), N, jnp.int32)])
    zpad = jnp.full((epad,), ZROW, jnp.int32)
    inph = [dst < PH, dst >= PH]
    srcp4 = jnp.concatenate(
        sum(
            [
                [jnp.where(inph[p], src + c * N, ZROW), zpad]
                for c in range(NC)
                for p in range(2)
            ],
            [],
        )
    )
    dstp2 = jnp.concatenate(
        [
            jnp.where(inph[0], dst, 0),
            jnp.zeros((epad,), jnp.int32),
            jnp.where(inph[1], dst - PH, 0),
            jnp.zeros((epad,), jnp.int32),
        ]
    )
    zo = jnp.concatenate(
        [
            jnp.zeros((NPAD // NS,), jnp.float32),
            jnp.ones((EPAD // (NC * NS),), jnp.float32),
        ]
    )

    degp = _make_degree_kernel()(dstd, zo)
    degt = degp.reshape(NC, NPAD)[:, :N].T

    tpad = ((0, TROWS - 2 * N), (0, 0))
    XC = _tc_prep(degt, x)
    prop = _make_prop_kernel()
    S1 = prop(jnp.pad(XC.reshape(NC * N, HALF), tpad), srcp4, dstp2)
    S1 = S1.reshape(NC, 2 * PH, HALF)[:, :N, :]
    Y2 = _tc_main(degt, S1, XC, W1, b1.reshape(1, -1), W2)
    S2 = prop(jnp.pad(Y2.reshape(NC * N, HALF), tpad), srcp4, dstp2)
    S2 = S2.reshape(NC, 2 * PH, HALF)[:, :N, :]
    return _tc_final(degt, S2, Y2, b2.reshape(1, -1))
